# Initial kernel scaffold; baseline (speedup 1.0000x reference)
import jax
import jax.numpy as jnp
from jax.experimental import pallas as pl


def kernel(x, edge_index, W_pre, b_pre, W1, b1, W2, b2):
    raise NotImplementedError("write your pallas kernel here")


# speedup vs baseline: 8.7107x; 8.7107x over previous
import functools

import jax
import jax.numpy as jnp
from jax import lax
from jax.experimental import pallas as pl
from jax.experimental.pallas import tpu as pltpu
from jax.experimental.pallas import tpu_sc as plsc

NC = 2
NS = 16
CH = 128


def _linear_body(x_ref, w_ref, b_ref, o_ref):
    o_ref[...] = (
        jnp.dot(x_ref[...], w_ref[...], preferred_element_type=jnp.float32)
        + b_ref[...]
    )


def _tc_linear(x, w, b, bn=400):
    n, d = x.shape
    grid = (n // bn,)
    return pl.pallas_call(
        _linear_body,
        grid=grid,
        in_specs=[
            pl.BlockSpec((bn, d), lambda i: (i, 0)),
            pl.BlockSpec((d, d), lambda i: (0, 0)),
            pl.BlockSpec((1, d), lambda i: (0, 0)),
        ],
        out_specs=pl.BlockSpec((bn, d), lambda i: (i, 0)),
        out_shape=jax.ShapeDtypeStruct((n, d), jnp.float32),
    )(x, w, b.reshape(1, d))


def _mean_linear_body(relu, normalize, p_ref, h_ref, d_ref, l_ref, w_ref,
                      b_ref, o_ref):
    deg = d_ref[0] + d_ref[1]
    loops = l_ref[0] + l_ref[1]
    coef = jnp.where(loops > 0.5, 0.0, 1.0)
    denom = jnp.maximum(deg + coef, 1.0)
    acc = p_ref[0] + p_ref[1] + coef * h_ref[...]
    mean = acc / denom
    y = (
        jnp.dot(mean, w_ref[...], preferred_element_type=jnp.float32)
        + b_ref[...]
    )
    if relu:
        y = jnp.maximum(y, 0.0)
    if normalize:
        nrm = jnp.sqrt(jnp.sum(y * y, axis=-1, keepdims=True))
        y = y / jnp.maximum(nrm, 1e-12)
    o_ref[...] = y


def _tc_mean_linear(p, h, deg3, loop3, w, b, relu, normalize, bn=400):
    n, d = h.shape
    grid = (n // bn,)
    body = functools.partial(_mean_linear_body, relu, normalize)
    return pl.pallas_call(
        body,
        grid=grid,
        in_specs=[
            pl.BlockSpec((NC, bn, d), lambda i: (0, i, 0)),
            pl.BlockSpec((bn, d), lambda i: (i, 0)),
            pl.BlockSpec((NC, bn, 1), lambda i: (0, i, 0)),
            pl.BlockSpec((NC, bn, 1), lambda i: (0, i, 0)),
            pl.BlockSpec((d, d), lambda i: (0, 0)),
            pl.BlockSpec((1, d), lambda i: (0, 0)),
        ],
        out_specs=pl.BlockSpec((bn, d), lambda i: (i, 0)),
        out_shape=jax.ShapeDtypeStruct((n, d), jnp.float32),
    )(p, h, deg3, loop3, w, b.reshape(1, d))


def _sc_aggregate(h, srcp, dstp, n_pad, with_counts):
    _, d = h.shape
    ep = srcp.shape[0]
    epw = ep // (NC * NS)
    rpt = n_pad // NS
    zr = 64

    mesh = plsc.VectorSubcoreMesh(core_axis_name="c", subcore_axis_name="s")

    out_type = [jax.ShapeDtypeStruct((NC, n_pad, d), jnp.float32)]
    scratch = [
        pltpu.VMEM((CH,), jnp.int32),
        pltpu.VMEM((CH,), jnp.int32),
        pltpu.VMEM((CH, d), jnp.float32),
        pltpu.VMEM((zr, d), jnp.float32),
        pltpu.VMEM_SHARED((n_pad, d), jnp.float32),
        pltpu.SemaphoreType.DMA,
    ]
    if with_counts:
        out_type += [
            jax.ShapeDtypeStruct((NC, n_pad), jnp.float32),
            jax.ShapeDtypeStruct((NC, n_pad), jnp.float32),
        ]
        scratch += [
            pltpu.VMEM((CH,), jnp.float32),
            pltpu.VMEM((CH,), jnp.float32),
            pltpu.VMEM((rpt,), jnp.float32),
            pltpu.VMEM_SHARED((n_pad,), jnp.float32),
            pltpu.VMEM_SHARED((n_pad,), jnp.float32),
        ]

    def body(*refs):
        if with_counts:
            (h_hbm, src_hbm, dst_hbm, out_hbm, deg_hbm, loop_hbm,
             src_v, dst_v, rows_v, zrows_v, facc, sem,
             ones_v, eq_v, zvec_v, dacc, lacc) = refs
        else:
            (h_hbm, src_hbm, dst_hbm, out_hbm,
             src_v, dst_v, rows_v, zrows_v, facc, sem) = refs

        c = lax.axis_index("c")
        s = lax.axis_index("s")
        wid = c * NS + s

        def zrow_body(i, carry):
            r = i // (d // 16)
            col = (i % (d // 16)) * 16
            zrows_v[r, pl.ds(col, 16)] = jnp.zeros((16,), jnp.float32)
            return carry
        lax.fori_loop(0, zr * (d // 16), zrow_body, 0)

        for t in range(rpt // zr):
            pltpu.sync_copy(zrows_v, facc.at[pl.ds(s * rpt + t * zr, zr)])
        if with_counts:
            def zvec_body(i, carry):
                zvec_v[pl.ds(i * 16, 16)] = jnp.zeros((16,), jnp.float32)
                return carry
            lax.fori_loop(0, rpt // 16, zvec_body, 0)
            pltpu.sync_copy(zvec_v, dacc.at[pl.ds(s * rpt, rpt)])
            pltpu.sync_copy(zvec_v, lacc.at[pl.ds(s * rpt, rpt)])
            for j in range(CH // 16):
                ones_v[pl.ds(j * 16, 16)] = jnp.ones((16,), jnp.float32)

        plsc.subcore_barrier()

        def chunk_body(k, carry):
            base = wid * epw + k * CH
            pltpu.sync_copy(src_hbm.at[pl.ds(base, CH)], src_v)
            pltpu.sync_copy(dst_hbm.at[pl.ds(base, CH)], dst_v)
            pltpu.async_copy(h_hbm.at[src_v], rows_v, sem).wait()
            pltpu.sync_copy(rows_v, facc.at[dst_v], add=True)
            if with_counts:
                pltpu.sync_copy(ones_v, dacc.at[dst_v], add=True)
                for j in range(CH // 16):
                    sl = pl.ds(j * 16, 16)
                    eq_v[sl] = jnp.where(src_v[sl] == dst_v[sl],
                                         jnp.float32(1.0), jnp.float32(0.0))
                pltpu.sync_copy(eq_v, lacc.at[src_v], add=True)
            return carry
        lax.fori_loop(0, epw // CH, chunk_body, 0)

        plsc.subcore_barrier()

        pltpu.sync_copy(facc.at[pl.ds(s * rpt, rpt)],
                        out_hbm.at[c, pl.ds(s * rpt, rpt)])
        if with_counts:
            pltpu.sync_copy(dacc.at[pl.ds(s * rpt, rpt)],
                            deg_hbm.at[c, pl.ds(s * rpt, rpt)])
            pltpu.sync_copy(lacc.at[pl.ds(s * rpt, rpt)],
                            loop_hbm.at[c, pl.ds(s * rpt, rpt)])

    fn = pl.kernel(body, out_type=out_type, scratch_types=scratch, mesh=mesh)
    return fn(h, srcp, dstp)


def kernel(x, edge_index, W_pre, b_pre, W1, b1, W2, b2):
    n, d = x.shape
    e = edge_index.shape[1]

    rpt = -((n + 1) // -(NS * 64)) * 64
    n_pad = NS * rpt
    ew = NC * NS * CH
    e_pad = -(e // -ew) * ew

    src = edge_index[0]
    dst = edge_index[1]
    pad = e_pad - e
    srcp = jnp.concatenate([src, jnp.zeros((pad,), jnp.int32)])
    dstp = jnp.concatenate([dst, jnp.full((pad,), n_pad - 1, jnp.int32)])

    h0 = _tc_linear(x, W_pre, b_pre)
    p1, deg, loops = _sc_aggregate(h0, srcp, dstp, n_pad, True)
    deg3 = deg.reshape(NC, n_pad, 1)
    loop3 = loops.reshape(NC, n_pad, 1)
    h1 = _tc_mean_linear(p1, h0, deg3, loop3, W1, b1, relu=True,
                         normalize=False)
    (p2,) = _sc_aggregate(h1, srcp, dstp, n_pad, False)
    return _tc_mean_linear(p2, h1, deg3, loop3, W2, b2, relu=False,
                           normalize=True)


# speedup vs baseline: 11.9599x; 1.3730x over previous
import functools

import jax
import jax.numpy as jnp
from jax import lax
from jax.experimental import pallas as pl
from jax.experimental.pallas import tpu as pltpu
from jax.experimental.pallas import tpu_sc as plsc

NC = 2
NS = 16
CH = 128


def _linear_body(x_ref, w_ref, b_ref, o_ref):
    o_ref[...] = (
        jnp.dot(x_ref[...], w_ref[...], preferred_element_type=jnp.float32)
        + b_ref[...]
    )


def _tc_linear(x, w, b, bn=400):
    n, d = x.shape
    grid = (n // bn,)
    return pl.pallas_call(
        _linear_body,
        grid=grid,
        in_specs=[
            pl.BlockSpec((bn, d), lambda i: (i, 0)),
            pl.BlockSpec((d, d), lambda i: (0, 0)),
            pl.BlockSpec((1, d), lambda i: (0, 0)),
        ],
        out_specs=pl.BlockSpec((bn, d), lambda i: (i, 0)),
        out_shape=jax.ShapeDtypeStruct((n, d), jnp.float32),
    )(x, w, b.reshape(1, d))


def _mean_linear_body(relu, normalize, p_ref, h_ref, d_ref, l_ref, w_ref,
                      b_ref, o_ref):
    deg = d_ref[0] + d_ref[1]
    loops = l_ref[0] + l_ref[1]
    coef = jnp.where(loops > 0.5, 0.0, 1.0)
    denom = jnp.maximum(deg + coef, 1.0)
    acc = p_ref[0] + p_ref[1] + coef * h_ref[...]
    mean = acc / denom
    y = (
        jnp.dot(mean, w_ref[...], preferred_element_type=jnp.float32)
        + b_ref[...]
    )
    if relu:
        y = jnp.maximum(y, 0.0)
    if normalize:
        nrm = jnp.sqrt(jnp.sum(y * y, axis=-1, keepdims=True))
        y = y / jnp.maximum(nrm, 1e-12)
    o_ref[...] = y


def _tc_mean_linear(p, h, deg3, loop3, w, b, relu, normalize, bn=400):
    n, d = h.shape
    grid = (n // bn,)
    body = functools.partial(_mean_linear_body, relu, normalize)
    return pl.pallas_call(
        body,
        grid=grid,
        in_specs=[
            pl.BlockSpec((NC, bn, d), lambda i: (0, i, 0)),
            pl.BlockSpec((bn, d), lambda i: (i, 0)),
            pl.BlockSpec((NC, bn, 1), lambda i: (0, i, 0)),
            pl.BlockSpec((NC, bn, 1), lambda i: (0, i, 0)),
            pl.BlockSpec((d, d), lambda i: (0, 0)),
            pl.BlockSpec((1, d), lambda i: (0, 0)),
        ],
        out_specs=pl.BlockSpec((bn, d), lambda i: (i, 0)),
        out_shape=jax.ShapeDtypeStruct((n, d), jnp.float32),
    )(p, h, deg3, loop3, w, b.reshape(1, d))


def _sc_aggregate(h, ei3, n_pad, with_counts):
    _, d = h.shape
    nch_tot = ei3.shape[0]
    nch = nch_tot // (NC * NS)
    rpt = n_pad // NS
    zr = 64

    mesh = plsc.VectorSubcoreMesh(core_axis_name="c", subcore_axis_name="s")

    out_type = [jax.ShapeDtypeStruct((NC, n_pad, d), jnp.float32)]
    scratch = [
        pltpu.VMEM((2, CH), jnp.int32),
        pltpu.VMEM((2, CH), jnp.int32),
        pltpu.VMEM((CH, d), jnp.float32),
        pltpu.VMEM((CH, d), jnp.float32),
        pltpu.VMEM((zr, d), jnp.float32),
        pltpu.VMEM_SHARED((n_pad, d), jnp.float32),
        pltpu.SemaphoreType.DMA,
        pltpu.SemaphoreType.DMA,
    ]
    if with_counts:
        out_type += [
            jax.ShapeDtypeStruct((NC, n_pad), jnp.float32),
            jax.ShapeDtypeStruct((NC, n_pad), jnp.float32),
        ]
        scratch += [
            pltpu.VMEM((CH,), jnp.float32),
            pltpu.VMEM((CH,), jnp.float32),
            pltpu.VMEM((rpt,), jnp.float32),
            pltpu.VMEM_SHARED((n_pad,), jnp.float32),
            pltpu.VMEM_SHARED((n_pad,), jnp.float32),
        ]

    def body(*refs):
        if with_counts:
            (h_hbm, ei_hbm, out_hbm, deg_hbm, loop_hbm,
             idx0, idx1, rows0, rows1, zrows_v, facc, sem0, sem1,
             ones_v, eq_v, zvec_v, dacc, lacc) = refs
        else:
            (h_hbm, ei_hbm, out_hbm,
             idx0, idx1, rows0, rows1, zrows_v, facc, sem0, sem1) = refs
        idx = (idx0, idx1)
        rows = (rows0, rows1)
        sems = (sem0, sem1)

        c = lax.axis_index("c")
        s = lax.axis_index("s")
        wid = c * NS + s
        ch0 = wid * nch

        def zrow_body(i, carry):
            r = i // (d // 16)
            col = (i % (d // 16)) * 16
            zrows_v[r, pl.ds(col, 16)] = jnp.zeros((16,), jnp.float32)
            return carry
        lax.fori_loop(0, zr * (d // 16), zrow_body, 0)

        for t in range(rpt // zr):
            pltpu.sync_copy(zrows_v, facc.at[pl.ds(s * rpt + t * zr, zr)])
        if with_counts:
            def zvec_body(i, carry):
                zvec_v[pl.ds(i * 16, 16)] = jnp.zeros((16,), jnp.float32)
                return carry
            lax.fori_loop(0, rpt // 16, zvec_body, 0)
            pltpu.sync_copy(zvec_v, dacc.at[pl.ds(s * rpt, rpt)])
            pltpu.sync_copy(zvec_v, lacc.at[pl.ds(s * rpt, rpt)])
            for j in range(CH // 16):
                ones_v[pl.ds(j * 16, 16)] = jnp.ones((16,), jnp.float32)

        plsc.subcore_barrier()

        def prefetch(k, b):
            pltpu.sync_copy(ei_hbm.at[ch0 + k], idx[b])
            pltpu.async_copy(h_hbm.at[idx[b].at[0]], rows[b], sems[b])

        def consume(k, b):
            pltpu.make_async_copy(h_hbm.at[idx[b].at[0]], rows[b],
                                  sems[b]).wait()
            pltpu.sync_copy(rows[b], facc.at[idx[b].at[1]], add=True)
            if with_counts:
                pltpu.sync_copy(ones_v, dacc.at[idx[b].at[1]], add=True)
                for j in range(CH // 16):
                    sl = pl.ds(j * 16, 16)
                    eq_v[sl] = jnp.where(idx[b][0, sl] == idx[b][1, sl],
                                         jnp.float32(1.0), jnp.float32(0.0))
                pltpu.sync_copy(eq_v, lacc.at[idx[b].at[0]], add=True)

        prefetch(0, 0)
        m = (nch - 1) // 2

        def pair_body(k2, carry):
            k = 2 * k2
            prefetch(k + 1, 1)
            consume(k, 0)
            prefetch(k + 2, 0)
            consume(k + 1, 1)
            return carry
        lax.fori_loop(0, m, pair_body, 0)

        for k in range(2 * m, nch):
            b = k % 2
            if k + 1 < nch:
                prefetch(k + 1, (k + 1) % 2)
            consume(k, b)

        plsc.subcore_barrier()

        pltpu.sync_copy(facc.at[pl.ds(s * rpt, rpt)],
                        out_hbm.at[c, pl.ds(s * rpt, rpt)])
        if with_counts:
            pltpu.sync_copy(dacc.at[pl.ds(s * rpt, rpt)],
                            deg_hbm.at[c, pl.ds(s * rpt, rpt)])
            pltpu.sync_copy(lacc.at[pl.ds(s * rpt, rpt)],
                            loop_hbm.at[c, pl.ds(s * rpt, rpt)])

    fn = pl.kernel(body, out_type=out_type, scratch_types=scratch, mesh=mesh)
    return fn(h, ei3)


def kernel(x, edge_index, W_pre, b_pre, W1, b1, W2, b2):
    n, d = x.shape
    e = edge_index.shape[1]

    rpt = -((n + 1) // -(NS * 64)) * 64
    n_pad = NS * rpt
    ew = NC * NS * CH
    e_pad = -(e // -ew) * ew

    src = edge_index[0]
    dst = edge_index[1]
    pad = e_pad - e
    srcp = jnp.concatenate([src, jnp.zeros((pad,), jnp.int32)])
    dstp = jnp.concatenate([dst, jnp.full((pad,), n_pad - 1, jnp.int32)])
    ei3 = jnp.stack([srcp.reshape(-1, CH), dstp.reshape(-1, CH)], axis=1)

    h0 = _tc_linear(x, W_pre, b_pre)
    p1, deg, loops = _sc_aggregate(h0, ei3, n_pad, True)
    deg3 = deg.reshape(NC, n_pad, 1)
    loop3 = loops.reshape(NC, n_pad, 1)
    h1 = _tc_mean_linear(p1, h0, deg3, loop3, W1, b1, relu=True,
                         normalize=False)
    (p2,) = _sc_aggregate(h1, ei3, n_pad, False)
    return _tc_mean_linear(p2, h1, deg3, loop3, W2, b2, relu=False,
                           normalize=True)


# speedup vs baseline: 13.1040x; 1.0957x over previous
import functools

import jax
import jax.numpy as jnp
from jax import lax
from jax.experimental import pallas as pl
from jax.experimental.pallas import tpu as pltpu
from jax.experimental.pallas import tpu_sc as plsc

NC = 2
NS = 16
CH = 128


def _linear_body(x_ref, w_ref, b_ref, o_ref):
    o_ref[...] = (
        jnp.dot(x_ref[...], w_ref[...], preferred_element_type=jnp.float32)
        + b_ref[...]
    )


def _tc_linear(x, w, b, bn=400):
    n, d = x.shape
    grid = (n // bn,)
    return pl.pallas_call(
        _linear_body,
        grid=grid,
        in_specs=[
            pl.BlockSpec((bn, d), lambda i: (i, 0)),
            pl.BlockSpec((d, d), lambda i: (0, 0)),
            pl.BlockSpec((1, d), lambda i: (0, 0)),
        ],
        out_specs=pl.BlockSpec((bn, d), lambda i: (i, 0)),
        out_shape=jax.ShapeDtypeStruct((n, d), jnp.float32),
    )(x, w, b.reshape(1, d))


def _mean_linear_body(relu, normalize, p_ref, h_ref, d_ref, l_ref, w_ref,
                      b_ref, o_ref):
    deg = d_ref[0] + d_ref[1]
    loops = l_ref[0] + l_ref[1]
    coef = jnp.where(loops > 0.5, 0.0, 1.0)
    denom = jnp.maximum(deg + coef, 1.0)
    acc = p_ref[0] + p_ref[1] + coef * h_ref[...]
    mean = acc / denom
    y = (
        jnp.dot(mean, w_ref[...], preferred_element_type=jnp.float32)
        + b_ref[...]
    )
    if relu:
        y = jnp.maximum(y, 0.0)
    if normalize:
        nrm = jnp.sqrt(jnp.sum(y * y, axis=-1, keepdims=True))
        y = y / jnp.maximum(nrm, 1e-12)
    o_ref[...] = y


def _tc_mean_linear(p, h, deg3, loop3, w, b, relu, normalize, bn=400):
    n, d = h.shape
    grid = (n // bn,)
    body = functools.partial(_mean_linear_body, relu, normalize)
    return pl.pallas_call(
        body,
        grid=grid,
        in_specs=[
            pl.BlockSpec((NC, bn, d), lambda i: (0, i, 0)),
            pl.BlockSpec((bn, d), lambda i: (i, 0)),
            pl.BlockSpec((NC, bn, 1), lambda i: (0, i, 0)),
            pl.BlockSpec((NC, bn, 1), lambda i: (0, i, 0)),
            pl.BlockSpec((d, d), lambda i: (0, 0)),
            pl.BlockSpec((1, d), lambda i: (0, 0)),
        ],
        out_specs=pl.BlockSpec((bn, d), lambda i: (i, 0)),
        out_shape=jax.ShapeDtypeStruct((n, d), jnp.float32),
    )(p, h, deg3, loop3, w, b.reshape(1, d))


def _sc_aggregate(h, ei3, n_pad, with_counts):
    _, d = h.shape
    nch_tot = ei3.shape[0]
    tot = nch_tot // NS
    n0 = max(2, int(round(tot * 0.68 / 2)) * 2)
    n1 = tot - n0
    rpt = n_pad // NS
    zr = 64

    mesh = plsc.VectorSubcoreMesh(core_axis_name="c", subcore_axis_name="s")

    out_type = [jax.ShapeDtypeStruct((NC, n_pad, d), jnp.float32)]
    scratch = [
        pltpu.VMEM((2, CH), jnp.int32),
        pltpu.VMEM((2, CH), jnp.int32),
        pltpu.VMEM((CH, d), jnp.float32),
        pltpu.VMEM((CH, d), jnp.float32),
        pltpu.VMEM((zr, d), jnp.float32),
        pltpu.VMEM_SHARED((n_pad, d), jnp.float32),
        pltpu.SemaphoreType.DMA,
        pltpu.SemaphoreType.DMA,
    ]
    if with_counts:
        out_type += [
            jax.ShapeDtypeStruct((NC, n_pad), jnp.float32),
            jax.ShapeDtypeStruct((NC, n_pad), jnp.float32),
        ]
        scratch += [
            pltpu.VMEM((CH,), jnp.float32),
            pltpu.VMEM((CH,), jnp.float32),
            pltpu.VMEM((rpt,), jnp.float32),
            pltpu.VMEM_SHARED((n_pad,), jnp.float32),
            pltpu.VMEM_SHARED((n_pad,), jnp.float32),
        ]

    def body(*refs):
        if with_counts:
            (h_hbm, ei_hbm, out_hbm, deg_hbm, loop_hbm,
             idx0, idx1, rows0, rows1, zrows_v, facc, sem0, sem1,
             ones_v, eq_v, zvec_v, dacc, lacc) = refs
        else:
            (h_hbm, ei_hbm, out_hbm,
             idx0, idx1, rows0, rows1, zrows_v, facc, sem0, sem1) = refs
        idx = (idx0, idx1)
        rows = (rows0, rows1)
        sems = (sem0, sem1)

        c = lax.axis_index("c")
        s = lax.axis_index("s")
        nch = jnp.where(c == 0, n0, n1)
        ch0 = jnp.where(c == 0, s * n0, NS * n0 + s * n1)

        def zrow_body(i, carry):
            r = i // (d // 16)
            col = (i % (d // 16)) * 16
            zrows_v[r, pl.ds(col, 16)] = jnp.zeros((16,), jnp.float32)
            return carry
        lax.fori_loop(0, zr * (d // 16), zrow_body, 0)

        for t in range(rpt // zr):
            pltpu.sync_copy(zrows_v, facc.at[pl.ds(s * rpt + t * zr, zr)])
        if with_counts:
            def zvec_body(i, carry):
                zvec_v[pl.ds(i * 16, 16)] = jnp.zeros((16,), jnp.float32)
                return carry
            lax.fori_loop(0, rpt // 16, zvec_body, 0)
            pltpu.sync_copy(zvec_v, dacc.at[pl.ds(s * rpt, rpt)])
            pltpu.sync_copy(zvec_v, lacc.at[pl.ds(s * rpt, rpt)])
            for j in range(CH // 16):
                ones_v[pl.ds(j * 16, 16)] = jnp.ones((16,), jnp.float32)

        plsc.subcore_barrier()

        def prefetch(k, b):
            pltpu.sync_copy(ei_hbm.at[ch0 + k], idx[b])
            pltpu.async_copy(h_hbm.at[idx[b].at[0]], rows[b], sems[b])

        def consume(k, b):
            pltpu.make_async_copy(h_hbm.at[idx[b].at[0]], rows[b],
                                  sems[b]).wait()
            pltpu.sync_copy(rows[b], facc.at[idx[b].at[1]], add=True)
            if with_counts:
                pltpu.sync_copy(ones_v, dacc.at[idx[b].at[1]], add=True)
                for j in range(CH // 16):
                    sl = pl.ds(j * 16, 16)
                    eq_v[sl] = jnp.where(idx[b][0, sl] == idx[b][1, sl],
                                         jnp.float32(1.0), jnp.float32(0.0))
                pltpu.sync_copy(eq_v, lacc.at[idx[b].at[0]], add=True)

        prefetch(0, 0)

        def pair_body(k2, carry):
            k = 2 * k2
            prefetch(k + 1, 1)
            consume(k, 0)

            @pl.when(k + 2 < nch)
            def _():
                prefetch(k + 2, 0)
            consume(k + 1, 1)
            return carry
        lax.fori_loop(0, nch // 2, pair_body, 0)

        plsc.subcore_barrier()

        pltpu.sync_copy(facc.at[pl.ds(s * rpt, rpt)],
                        out_hbm.at[c, pl.ds(s * rpt, rpt)])
        if with_counts:
            pltpu.sync_copy(dacc.at[pl.ds(s * rpt, rpt)],
                            deg_hbm.at[c, pl.ds(s * rpt, rpt)])
            pltpu.sync_copy(lacc.at[pl.ds(s * rpt, rpt)],
                            loop_hbm.at[c, pl.ds(s * rpt, rpt)])

    fn = pl.kernel(body, out_type=out_type, scratch_types=scratch, mesh=mesh)
    return fn(h, ei3)


def kernel(x, edge_index, W_pre, b_pre, W1, b1, W2, b2):
    n, d = x.shape
    e = edge_index.shape[1]

    rpt = -((n + 1) // -(NS * 64)) * 64
    n_pad = NS * rpt
    ew = NC * NS * CH
    e_pad = -(e // -ew) * ew

    src = edge_index[0]
    dst = edge_index[1]
    pad = e_pad - e
    srcp = jnp.concatenate([src, jnp.zeros((pad,), jnp.int32)])
    dstp = jnp.concatenate([dst, jnp.full((pad,), n_pad - 1, jnp.int32)])
    ei3 = jnp.stack([srcp.reshape(-1, CH), dstp.reshape(-1, CH)], axis=1)

    h0 = _tc_linear(x, W_pre, b_pre)
    p1, deg, loops = _sc_aggregate(h0, ei3, n_pad, True)
    deg3 = deg.reshape(NC, n_pad, 1)
    loop3 = loops.reshape(NC, n_pad, 1)
    h1 = _tc_mean_linear(p1, h0, deg3, loop3, W1, b1, relu=True,
                         normalize=False)
    (p2,) = _sc_aggregate(h1, ei3, n_pad, False)
    return _tc_mean_linear(p2, h1, deg3, loop3, W2, b2, relu=False,
                           normalize=True)


# speedup vs baseline: 13.1932x; 1.0068x over previous
import functools

import jax
import jax.numpy as jnp
from jax import lax
from jax.experimental import pallas as pl
from jax.experimental.pallas import tpu as pltpu
from jax.experimental.pallas import tpu_sc as plsc

NC = 2
NS = 16
CH = 128
NB = 2


def _linear_body(x_ref, w_ref, b_ref, o_ref):
    o_ref[...] = (
        jnp.dot(x_ref[...], w_ref[...], preferred_element_type=jnp.float32)
        + b_ref[...]
    )


def _tc_linear(x, w, b, bn=400):
    n, d = x.shape
    grid = (n // bn,)
    return pl.pallas_call(
        _linear_body,
        grid=grid,
        in_specs=[
            pl.BlockSpec((bn, d), lambda i: (i, 0)),
            pl.BlockSpec((d, d), lambda i: (0, 0)),
            pl.BlockSpec((1, d), lambda i: (0, 0)),
        ],
        out_specs=pl.BlockSpec((bn, d), lambda i: (i, 0)),
        out_shape=jax.ShapeDtypeStruct((n, d), jnp.float32),
    )(x, w, b.reshape(1, d))


def _mean_linear_body(relu, normalize, p_ref, h_ref, d_ref, l_ref, w_ref,
                      b_ref, o_ref):
    deg = d_ref[0] + d_ref[1]
    loops = l_ref[0] + l_ref[1]
    coef = jnp.where(loops > 0.5, 0.0, 1.0)
    denom = jnp.maximum(deg + coef, 1.0)
    acc = p_ref[0] + p_ref[1] + coef * h_ref[...]
    mean = acc / denom
    y = (
        jnp.dot(mean, w_ref[...], preferred_element_type=jnp.float32)
        + b_ref[...]
    )
    if relu:
        y = jnp.maximum(y, 0.0)
    if normalize:
        nrm = jnp.sqrt(jnp.sum(y * y, axis=-1, keepdims=True))
        y = y / jnp.maximum(nrm, 1e-12)
    o_ref[...] = y


def _tc_mean_linear(p, h, deg3, loop3, w, b, relu, normalize, bn=400):
    n, d = h.shape
    grid = (n // bn,)
    body = functools.partial(_mean_linear_body, relu, normalize)
    return pl.pallas_call(
        body,
        grid=grid,
        in_specs=[
            pl.BlockSpec((NC, bn, d), lambda i: (0, i, 0)),
            pl.BlockSpec((bn, d), lambda i: (i, 0)),
            pl.BlockSpec((NC, bn, 1), lambda i: (0, i, 0)),
            pl.BlockSpec((NC, bn, 1), lambda i: (0, i, 0)),
            pl.BlockSpec((d, d), lambda i: (0, 0)),
            pl.BlockSpec((1, d), lambda i: (0, 0)),
        ],
        out_specs=pl.BlockSpec((bn, d), lambda i: (i, 0)),
        out_shape=jax.ShapeDtypeStruct((n, d), jnp.float32),
    )(p, h, deg3, loop3, w, b.reshape(1, d))


def _sc_aggregate(h, ei3, n_pad, with_counts):
    _, d = h.shape
    nch_tot = ei3.shape[0]
    tot = nch_tot // NS
    n0 = max(NB, int(round(tot * 0.68 / NB)) * NB)
    n1 = tot - n0
    rpt = n_pad // NS
    zr = 64

    mesh = plsc.VectorSubcoreMesh(core_axis_name="c", subcore_axis_name="s")

    out_type = [jax.ShapeDtypeStruct((NC, n_pad, d), jnp.float32)]
    scratch = (
        [pltpu.VMEM((2, CH), jnp.int32) for _ in range(NB)]
        + [pltpu.VMEM((CH, d), jnp.float32) for _ in range(NB)]
        + [pltpu.VMEM((zr, d), jnp.float32)]
        + [pltpu.VMEM_SHARED((n_pad, d), jnp.float32)]
        + [pltpu.SemaphoreType.DMA for _ in range(2 * NB)]
    )
    if with_counts:
        out_type += [
            jax.ShapeDtypeStruct((NC, n_pad), jnp.float32),
            jax.ShapeDtypeStruct((NC, n_pad), jnp.float32),
        ]
        scratch += (
            [pltpu.VMEM((CH,), jnp.float32) for _ in range(NB)]
            + [pltpu.VMEM((CH,), jnp.float32)]
            + [pltpu.VMEM((rpt,), jnp.float32)]
            + [pltpu.VMEM_SHARED((n_pad,), jnp.float32)]
            + [pltpu.VMEM_SHARED((n_pad,), jnp.float32)]
            + [pltpu.SemaphoreType.DMA for _ in range(2 * NB)]
        )

    def body(*refs):
        if with_counts:
            (h_hbm, ei_hbm, out_hbm, deg_hbm, loop_hbm) = refs[:5]
            refs = refs[5:]
        else:
            (h_hbm, ei_hbm, out_hbm) = refs[:3]
            refs = refs[3:]
        idx = refs[:NB]
        rows = refs[NB:2 * NB]
        zrows_v = refs[2 * NB]
        facc = refs[2 * NB + 1]
        semg = refs[2 * NB + 2:3 * NB + 2]
        sems = refs[3 * NB + 2:4 * NB + 2]
        if with_counts:
            refs = refs[4 * NB + 2:]
            eqs = refs[:NB]
            ones_v = refs[NB]
            zvec_v = refs[NB + 1]
            dacc = refs[NB + 2]
            lacc = refs[NB + 3]
            semd = refs[NB + 4:2 * NB + 4]
            seml = refs[2 * NB + 4:3 * NB + 4]

        c = lax.axis_index("c")
        s = lax.axis_index("s")
        nch = jnp.where(c == 0, n0, n1)
        ch0 = jnp.where(c == 0, s * n0, NS * n0 + s * n1)

        def zrow_body(i, carry):
            r = i // (d // 16)
            col = (i % (d // 16)) * 16
            zrows_v[r, pl.ds(col, 16)] = jnp.zeros((16,), jnp.float32)
            return carry
        lax.fori_loop(0, zr * (d // 16), zrow_body, 0)

        for t in range(rpt // zr):
            pltpu.sync_copy(zrows_v, facc.at[pl.ds(s * rpt + t * zr, zr)])
        if with_counts:
            def zvec_body(i, carry):
                zvec_v[pl.ds(i * 16, 16)] = jnp.zeros((16,), jnp.float32)
                return carry
            lax.fori_loop(0, rpt // 16, zvec_body, 0)
            pltpu.sync_copy(zvec_v, dacc.at[pl.ds(s * rpt, rpt)])
            pltpu.sync_copy(zvec_v, lacc.at[pl.ds(s * rpt, rpt)])
            for j in range(CH // 16):
                ones_v[pl.ds(j * 16, 16)] = jnp.ones((16,), jnp.float32)

        plsc.subcore_barrier()

        def stage(k, b):
            pltpu.sync_copy(ei_hbm.at[ch0 + k], idx[b])
            pltpu.async_copy(h_hbm.at[idx[b].at[0]], rows[b], semg[b])

        def drain_scatters(b):
            pltpu.make_async_copy(rows[b], facc.at[idx[b].at[1]],
                                  sems[b]).wait()
            if with_counts:
                pltpu.make_async_copy(ones_v, dacc.at[idx[b].at[1]],
                                      semd[b]).wait()
                pltpu.make_async_copy(eqs[b], lacc.at[idx[b].at[0]],
                                      seml[b]).wait()

        def issue_scatters(b):
            pltpu.make_async_copy(h_hbm.at[idx[b].at[0]], rows[b],
                                  semg[b]).wait()
            pltpu.async_copy(rows[b], facc.at[idx[b].at[1]], sems[b],
                             add=True)
            if with_counts:
                pltpu.async_copy(ones_v, dacc.at[idx[b].at[1]], semd[b],
                                 add=True)
                for j in range(CH // 16):
                    sl = pl.ds(j * 16, 16)
                    eqs[b][sl] = jnp.where(idx[b][0, sl] == idx[b][1, sl],
                                           jnp.float32(1.0), jnp.float32(0.0))
                pltpu.async_copy(eqs[b], lacc.at[idx[b].at[0]], seml[b],
                                 add=True)

        def pair_body(k2, carry):
            @pl.when(k2 >= 1)
            def _():
                drain_scatters(0)
            stage(2 * k2, 0)

            @pl.when(k2 >= 1)
            def _():
                issue_scatters(1)
                drain_scatters(1)
            stage(2 * k2 + 1, 1)
            issue_scatters(0)
            return carry
        lax.fori_loop(0, nch // 2, pair_body, 0)

        issue_scatters(1)
        drain_scatters(0)
        drain_scatters(1)

        plsc.subcore_barrier()

        pltpu.sync_copy(facc.at[pl.ds(s * rpt, rpt)],
                        out_hbm.at[c, pl.ds(s * rpt, rpt)])
        if with_counts:
            pltpu.sync_copy(dacc.at[pl.ds(s * rpt, rpt)],
                            deg_hbm.at[c, pl.ds(s * rpt, rpt)])
            pltpu.sync_copy(lacc.at[pl.ds(s * rpt, rpt)],
                            loop_hbm.at[c, pl.ds(s * rpt, rpt)])

    fn = pl.kernel(body, out_type=out_type, scratch_types=scratch, mesh=mesh)
    return fn(h, ei3)


def kernel(x, edge_index, W_pre, b_pre, W1, b1, W2, b2):
    n, d = x.shape
    e = edge_index.shape[1]

    rpt = -((n + 1) // -(NS * 64)) * 64
    n_pad = NS * rpt
    ew = NB * NS * CH
    e_pad = -(e // -ew) * ew

    src = edge_index[0]
    dst = edge_index[1]
    pad = e_pad - e
    srcp = jnp.concatenate([src, jnp.zeros((pad,), jnp.int32)])
    dstp = jnp.concatenate([dst, jnp.full((pad,), n_pad - 1, jnp.int32)])
    ei3 = jnp.stack([srcp.reshape(-1, CH), dstp.reshape(-1, CH)], axis=1)

    h0 = _tc_linear(x, W_pre, b_pre)
    p1, deg, loops = _sc_aggregate(h0, ei3, n_pad, True)
    deg3 = deg.reshape(NC, n_pad, 1)
    loop3 = loops.reshape(NC, n_pad, 1)
    h1 = _tc_mean_linear(p1, h0, deg3, loop3, W1, b1, relu=True,
                         normalize=False)
    (p2,) = _sc_aggregate(h1, ei3, n_pad, False)
    return _tc_mean_linear(p2, h1, deg3, loop3, W2, b2, relu=False,
                           normalize=True)


# speedup vs baseline: 13.5562x; 1.0275x over previous
import functools

import jax
import jax.numpy as jnp
from jax import lax
from jax.experimental import pallas as pl
from jax.experimental.pallas import tpu as pltpu
from jax.experimental.pallas import tpu_sc as plsc

NC = 2
NS = 16
CH = 128
NB = 2


def _linear_body(x_ref, w_ref, b_ref, o_ref):
    o_ref[...] = (
        jnp.dot(x_ref[...], w_ref[...], preferred_element_type=jnp.float32)
        + b_ref[...]
    )


def _tc_linear(x, w, b, bn=400):
    n, d = x.shape
    grid = (n // bn,)
    return pl.pallas_call(
        _linear_body,
        grid=grid,
        in_specs=[
            pl.BlockSpec((bn, d), lambda i: (i, 0)),
            pl.BlockSpec((d, d), lambda i: (0, 0)),
            pl.BlockSpec((1, d), lambda i: (0, 0)),
        ],
        out_specs=pl.BlockSpec((bn, d), lambda i: (i, 0)),
        out_shape=jax.ShapeDtypeStruct((n, d), jnp.float32),
    )(x, w, b.reshape(1, d))


def _mean_linear_body(relu, normalize, p_ref, h_ref, d_ref, l_ref, w_ref,
                      b_ref, o_ref):
    deg = d_ref[0] + d_ref[1]
    loops = l_ref[0] + l_ref[1]
    coef = jnp.where(loops > 0.5, 0.0, 1.0)
    denom = jnp.maximum(deg + coef, 1.0)
    acc = p_ref[0] + p_ref[1] + coef * h_ref[...]
    mean = acc / denom
    y = (
        jnp.dot(mean, w_ref[...], preferred_element_type=jnp.float32)
        + b_ref[...]
    )
    if relu:
        y = jnp.maximum(y, 0.0)
    if normalize:
        nrm = jnp.sqrt(jnp.sum(y * y, axis=-1, keepdims=True))
        y = y / jnp.maximum(nrm, 1e-12)
    o_ref[...] = y


def _tc_mean_linear(p, h, deg3, loop3, w, b, relu, normalize, bn=400):
    n, d = h.shape
    grid = (n // bn,)
    body = functools.partial(_mean_linear_body, relu, normalize)
    return pl.pallas_call(
        body,
        grid=grid,
        in_specs=[
            pl.BlockSpec((NC, bn, d), lambda i: (0, i, 0)),
            pl.BlockSpec((bn, d), lambda i: (i, 0)),
            pl.BlockSpec((NC, bn, 1), lambda i: (0, i, 0)),
            pl.BlockSpec((NC, bn, 1), lambda i: (0, i, 0)),
            pl.BlockSpec((d, d), lambda i: (0, 0)),
            pl.BlockSpec((1, d), lambda i: (0, 0)),
        ],
        out_specs=pl.BlockSpec((bn, d), lambda i: (i, 0)),
        out_shape=jax.ShapeDtypeStruct((n, d), jnp.float32),
    )(p, h, deg3, loop3, w, b.reshape(1, d))


def _sc_aggregate(h, ei3, n_pad, with_counts):
    _, d = h.shape
    nch_tot = ei3.shape[0]
    tot = nch_tot // NS
    n0 = max(NB, int(round(tot * 0.735 / NB)) * NB)
    n1 = tot - n0
    rpt = n_pad // NS
    zr = 64

    mesh = plsc.VectorSubcoreMesh(core_axis_name="c", subcore_axis_name="s")

    out_type = [jax.ShapeDtypeStruct((NC, n_pad, d), jnp.float32)]
    scratch = (
        [pltpu.VMEM((2, CH), jnp.int32) for _ in range(NB)]
        + [pltpu.VMEM((CH, d), jnp.float32) for _ in range(NB)]
        + [pltpu.VMEM((zr, d), jnp.float32)]
        + [pltpu.VMEM_SHARED((n_pad, d), jnp.float32)]
        + [pltpu.SemaphoreType.DMA for _ in range(2 * NB)]
    )
    if with_counts:
        out_type += [
            jax.ShapeDtypeStruct((NC, n_pad), jnp.float32),
            jax.ShapeDtypeStruct((NC, n_pad), jnp.float32),
        ]
        scratch += (
            [pltpu.VMEM((CH,), jnp.float32) for _ in range(NB)]
            + [pltpu.VMEM((CH,), jnp.float32)]
            + [pltpu.VMEM((rpt,), jnp.float32)]
            + [pltpu.VMEM_SHARED((n_pad,), jnp.float32)]
            + [pltpu.VMEM_SHARED((n_pad,), jnp.float32)]
            + [pltpu.SemaphoreType.DMA for _ in range(2 * NB)]
        )

    def body(*refs):
        if with_counts:
            (h_hbm, ei_hbm, out_hbm, deg_hbm, loop_hbm) = refs[:5]
            refs = refs[5:]
        else:
            (h_hbm, ei_hbm, out_hbm) = refs[:3]
            refs = refs[3:]
        idx = refs[:NB]
        rows = refs[NB:2 * NB]
        zrows_v = refs[2 * NB]
        facc = refs[2 * NB + 1]
        semg = refs[2 * NB + 2:3 * NB + 2]
        sems = refs[3 * NB + 2:4 * NB + 2]
        if with_counts:
            refs = refs[4 * NB + 2:]
            eqs = refs[:NB]
            ones_v = refs[NB]
            zvec_v = refs[NB + 1]
            dacc = refs[NB + 2]
            lacc = refs[NB + 3]
            semd = refs[NB + 4:2 * NB + 4]
            seml = refs[2 * NB + 4:3 * NB + 4]

        c = lax.axis_index("c")
        s = lax.axis_index("s")
        nch = jnp.where(c == 0, n0, n1)
        ch0 = jnp.where(c == 0, s * n0, NS * n0 + s * n1)

        def zrow_body(i, carry):
            r = i // (d // 16)
            col = (i % (d // 16)) * 16
            zrows_v[r, pl.ds(col, 16)] = jnp.zeros((16,), jnp.float32)
            return carry
        lax.fori_loop(0, zr * (d // 16), zrow_body, 0)

        for t in range(rpt // zr):
            pltpu.sync_copy(zrows_v, facc.at[pl.ds(s * rpt + t * zr, zr)])
        if with_counts:
            def zvec_body(i, carry):
                zvec_v[pl.ds(i * 16, 16)] = jnp.zeros((16,), jnp.float32)
                return carry
            lax.fori_loop(0, rpt // 16, zvec_body, 0)
            pltpu.sync_copy(zvec_v, dacc.at[pl.ds(s * rpt, rpt)])
            pltpu.sync_copy(zvec_v, lacc.at[pl.ds(s * rpt, rpt)])
            for j in range(CH // 16):
                ones_v[pl.ds(j * 16, 16)] = jnp.ones((16,), jnp.float32)

        plsc.subcore_barrier()

        def stage(k, b):
            pltpu.sync_copy(ei_hbm.at[ch0 + k], idx[b])
            pltpu.async_copy(h_hbm.at[idx[b].at[0]], rows[b], semg[b])

        def drain_scatters(b):
            pltpu.make_async_copy(rows[b], facc.at[idx[b].at[1]],
                                  sems[b]).wait()
            if with_counts:
                pltpu.make_async_copy(ones_v, dacc.at[idx[b].at[1]],
                                      semd[b]).wait()
                pltpu.make_async_copy(eqs[b], lacc.at[idx[b].at[0]],
                                      seml[b]).wait()

        def issue_scatters(b):
            pltpu.make_async_copy(h_hbm.at[idx[b].at[0]], rows[b],
                                  semg[b]).wait()
            pltpu.async_copy(rows[b], facc.at[idx[b].at[1]], sems[b],
                             add=True)
            if with_counts:
                pltpu.async_copy(ones_v, dacc.at[idx[b].at[1]], semd[b],
                                 add=True)
                for j in range(CH // 16):
                    sl = pl.ds(j * 16, 16)
                    eqs[b][sl] = jnp.where(idx[b][0, sl] == idx[b][1, sl],
                                           jnp.float32(1.0), jnp.float32(0.0))
                pltpu.async_copy(eqs[b], lacc.at[idx[b].at[0]], seml[b],
                                 add=True)

        def pair_body(k2, carry):
            @pl.when(k2 >= 1)
            def _():
                drain_scatters(0)
            stage(2 * k2, 0)

            @pl.when(k2 >= 1)
            def _():
                issue_scatters(1)
                drain_scatters(1)
            stage(2 * k2 + 1, 1)
            issue_scatters(0)
            return carry
        lax.fori_loop(0, nch // 2, pair_body, 0)

        issue_scatters(1)
        drain_scatters(0)
        drain_scatters(1)

        plsc.subcore_barrier()

        pltpu.sync_copy(facc.at[pl.ds(s * rpt, rpt)],
                        out_hbm.at[c, pl.ds(s * rpt, rpt)])
        if with_counts:
            pltpu.sync_copy(dacc.at[pl.ds(s * rpt, rpt)],
                            deg_hbm.at[c, pl.ds(s * rpt, rpt)])
            pltpu.sync_copy(lacc.at[pl.ds(s * rpt, rpt)],
                            loop_hbm.at[c, pl.ds(s * rpt, rpt)])

    fn = pl.kernel(body, out_type=out_type, scratch_types=scratch, mesh=mesh)
    return fn(h, ei3)


def kernel(x, edge_index, W_pre, b_pre, W1, b1, W2, b2):
    n, d = x.shape
    e = edge_index.shape[1]

    rpt = -((n + 1) // -(NS * 64)) * 64
    n_pad = NS * rpt
    ew = NB * NS * CH
    e_pad = -(e // -ew) * ew

    src = edge_index[0]
    dst = edge_index[1]
    pad = e_pad - e
    srcp = jnp.concatenate([src, jnp.zeros((pad,), jnp.int32)])
    dstp = jnp.concatenate([dst, jnp.full((pad,), n_pad - 1, jnp.int32)])
    ei3 = jnp.stack([srcp.reshape(-1, CH), dstp.reshape(-1, CH)], axis=1)

    h0 = _tc_linear(x, W_pre, b_pre)
    p1, deg, loops = _sc_aggregate(h0, ei3, n_pad, True)
    deg3 = deg.reshape(NC, n_pad, 1)
    loop3 = loops.reshape(NC, n_pad, 1)
    h1 = _tc_mean_linear(p1, h0, deg3, loop3, W1, b1, relu=True,
                         normalize=False)
    (p2,) = _sc_aggregate(h1, ei3, n_pad, False)
    return _tc_mean_linear(p2, h1, deg3, loop3, W2, b2, relu=False,
                           normalize=True)


# speedup vs baseline: 14.3243x; 1.0567x over previous
import functools

import jax
import jax.numpy as jnp
from jax import lax
from jax.experimental import pallas as pl
from jax.experimental.pallas import tpu as pltpu
from jax.experimental.pallas import tpu_sc as plsc

NC = 2
NS = 16
CH = 128
NB = 2


def _linear_body(x_ref, w_ref, b_ref, o_ref):
    o_ref[...] = (
        jnp.dot(x_ref[...], w_ref[...], preferred_element_type=jnp.float32)
        + b_ref[...]
    )


def _tc_linear(x, w, b, bn=2000):
    n, d = x.shape
    grid = (n // bn,)
    return pl.pallas_call(
        _linear_body,
        grid=grid,
        in_specs=[
            pl.BlockSpec((bn, d), lambda i: (i, 0)),
            pl.BlockSpec((d, d), lambda i: (0, 0)),
            pl.BlockSpec((1, d), lambda i: (0, 0)),
        ],
        out_specs=pl.BlockSpec((bn, d), lambda i: (i, 0)),
        out_shape=jax.ShapeDtypeStruct((n, d), jnp.float32),
    )(x, w, b.reshape(1, d))


def _mean_linear_body(relu, normalize, p_ref, h_ref, d_ref, l_ref, w_ref,
                      b_ref, o_ref):
    deg = d_ref[0] + d_ref[1]
    loops = l_ref[0] + l_ref[1]
    coef = jnp.where(loops > 0.5, 0.0, 1.0)
    denom = jnp.maximum(deg + coef, 1.0)
    acc = p_ref[0] + p_ref[1] + coef * h_ref[...]
    mean = acc / denom
    y = (
        jnp.dot(mean, w_ref[...], preferred_element_type=jnp.float32)
        + b_ref[...]
    )
    if relu:
        y = jnp.maximum(y, 0.0)
    if normalize:
        nrm = jnp.sqrt(jnp.sum(y * y, axis=-1, keepdims=True))
        y = y / jnp.maximum(nrm, 1e-12)
    o_ref[...] = y


def _tc_mean_linear(p, h, deg3, loop3, w, b, relu, normalize, bn=2000):
    n, d = h.shape
    grid = (n // bn,)
    body = functools.partial(_mean_linear_body, relu, normalize)
    return pl.pallas_call(
        body,
        grid=grid,
        in_specs=[
            pl.BlockSpec((NC, bn, d), lambda i: (0, i, 0)),
            pl.BlockSpec((bn, d), lambda i: (i, 0)),
            pl.BlockSpec((NC, bn, 1), lambda i: (0, i, 0)),
            pl.BlockSpec((NC, bn, 1), lambda i: (0, i, 0)),
            pl.BlockSpec((d, d), lambda i: (0, 0)),
            pl.BlockSpec((1, d), lambda i: (0, 0)),
        ],
        out_specs=pl.BlockSpec((bn, d), lambda i: (i, 0)),
        out_shape=jax.ShapeDtypeStruct((n, d), jnp.float32),
    )(p, h, deg3, loop3, w, b.reshape(1, d))


def _sc_aggregate(h, ei3, n_pad, with_counts):
    _, d = h.shape
    nch_tot = ei3.shape[0]
    tot = nch_tot // NS
    n0 = max(NB, int(round(tot * 0.735 / NB)) * NB)
    n1 = tot - n0
    rpt = n_pad // NS
    zr = 64

    mesh = plsc.VectorSubcoreMesh(core_axis_name="c", subcore_axis_name="s")

    out_type = [jax.ShapeDtypeStruct((NC, n_pad, d), jnp.float32)]
    scratch = (
        [pltpu.VMEM((2, CH), jnp.int32) for _ in range(NB)]
        + [pltpu.VMEM((CH, d), jnp.float32) for _ in range(NB)]
        + [pltpu.VMEM((zr, d), jnp.float32)]
        + [pltpu.VMEM_SHARED((n_pad, d), jnp.float32)]
        + [pltpu.SemaphoreType.DMA for _ in range(2 * NB)]
    )
    if with_counts:
        out_type += [
            jax.ShapeDtypeStruct((NC, n_pad), jnp.float32),
            jax.ShapeDtypeStruct((NC, n_pad), jnp.float32),
        ]
        scratch += (
            [pltpu.VMEM((CH,), jnp.float32) for _ in range(NB)]
            + [pltpu.VMEM((CH,), jnp.float32)]
            + [pltpu.VMEM((rpt,), jnp.float32)]
            + [pltpu.VMEM_SHARED((n_pad,), jnp.float32)]
            + [pltpu.VMEM_SHARED((n_pad,), jnp.float32)]
            + [pltpu.SemaphoreType.DMA for _ in range(2 * NB)]
        )

    def body(*refs):
        if with_counts:
            (h_hbm, ei_hbm, out_hbm, deg_hbm, loop_hbm) = refs[:5]
            refs = refs[5:]
        else:
            (h_hbm, ei_hbm, out_hbm) = refs[:3]
            refs = refs[3:]
        idx = refs[:NB]
        rows = refs[NB:2 * NB]
        zrows_v = refs[2 * NB]
        facc = refs[2 * NB + 1]
        semg = refs[2 * NB + 2:3 * NB + 2]
        sems = refs[3 * NB + 2:4 * NB + 2]
        if with_counts:
            refs = refs[4 * NB + 2:]
            eqs = refs[:NB]
            ones_v = refs[NB]
            zvec_v = refs[NB + 1]
            dacc = refs[NB + 2]
            lacc = refs[NB + 3]
            semd = refs[NB + 4:2 * NB + 4]
            seml = refs[2 * NB + 4:3 * NB + 4]

        c = lax.axis_index("c")
        s = lax.axis_index("s")
        nch = jnp.where(c == 0, n0, n1)
        ch0 = jnp.where(c == 0, s * n0, NS * n0 + s * n1)

        def zrow_body(i, carry):
            r = i // (d // 16)
            col = (i % (d // 16)) * 16
            zrows_v[r, pl.ds(col, 16)] = jnp.zeros((16,), jnp.float32)
            return carry
        lax.fori_loop(0, zr * (d // 16), zrow_body, 0)

        for t in range(rpt // zr):
            pltpu.sync_copy(zrows_v, facc.at[pl.ds(s * rpt + t * zr, zr)])
        if with_counts:
            def zvec_body(i, carry):
                zvec_v[pl.ds(i * 16, 16)] = jnp.zeros((16,), jnp.float32)
                return carry
            lax.fori_loop(0, rpt // 16, zvec_body, 0)
            pltpu.sync_copy(zvec_v, dacc.at[pl.ds(s * rpt, rpt)])
            pltpu.sync_copy(zvec_v, lacc.at[pl.ds(s * rpt, rpt)])
            for j in range(CH // 16):
                ones_v[pl.ds(j * 16, 16)] = jnp.ones((16,), jnp.float32)

        plsc.subcore_barrier()

        def stage(k, b):
            pltpu.sync_copy(ei_hbm.at[ch0 + k], idx[b])
            pltpu.async_copy(h_hbm.at[idx[b].at[0]], rows[b], semg[b])

        def drain_scatters(b):
            pltpu.make_async_copy(rows[b], facc.at[idx[b].at[1]],
                                  sems[b]).wait()
            if with_counts:
                pltpu.make_async_copy(ones_v, dacc.at[idx[b].at[1]],
                                      semd[b]).wait()
                pltpu.make_async_copy(eqs[b], lacc.at[idx[b].at[0]],
                                      seml[b]).wait()

        def issue_scatters(b):
            pltpu.make_async_copy(h_hbm.at[idx[b].at[0]], rows[b],
                                  semg[b]).wait()
            pltpu.async_copy(rows[b], facc.at[idx[b].at[1]], sems[b],
                             add=True)
            if with_counts:
                pltpu.async_copy(ones_v, dacc.at[idx[b].at[1]], semd[b],
                                 add=True)
                for j in range(CH // 16):
                    sl = pl.ds(j * 16, 16)
                    eqs[b][sl] = jnp.where(idx[b][0, sl] == idx[b][1, sl],
                                           jnp.float32(1.0), jnp.float32(0.0))
                pltpu.async_copy(eqs[b], lacc.at[idx[b].at[0]], seml[b],
                                 add=True)

        def pair_body(k2, carry):
            @pl.when(k2 >= 1)
            def _():
                drain_scatters(0)
            stage(2 * k2, 0)

            @pl.when(k2 >= 1)
            def _():
                issue_scatters(1)
                drain_scatters(1)
            stage(2 * k2 + 1, 1)
            issue_scatters(0)
            return carry
        lax.fori_loop(0, nch // 2, pair_body, 0)

        issue_scatters(1)
        drain_scatters(0)
        drain_scatters(1)

        plsc.subcore_barrier()

        pltpu.sync_copy(facc.at[pl.ds(s * rpt, rpt)],
                        out_hbm.at[c, pl.ds(s * rpt, rpt)])
        if with_counts:
            pltpu.sync_copy(dacc.at[pl.ds(s * rpt, rpt)],
                            deg_hbm.at[c, pl.ds(s * rpt, rpt)])
            pltpu.sync_copy(lacc.at[pl.ds(s * rpt, rpt)],
                            loop_hbm.at[c, pl.ds(s * rpt, rpt)])

    fn = pl.kernel(body, out_type=out_type, scratch_types=scratch, mesh=mesh)
    return fn(h, ei3)


def kernel(x, edge_index, W_pre, b_pre, W1, b1, W2, b2):
    n, d = x.shape
    e = edge_index.shape[1]

    rpt = -((n + 1) // -(NS * 64)) * 64
    n_pad = NS * rpt
    ew = NB * NS * CH
    e_pad = -(e // -ew) * ew

    src = edge_index[0]
    dst = edge_index[1]
    pad = e_pad - e
    srcp = jnp.concatenate([src, jnp.zeros((pad,), jnp.int32)])
    dstp = jnp.concatenate([dst, jnp.full((pad,), n_pad - 1, jnp.int32)])
    ei3 = jnp.stack([srcp.reshape(-1, CH), dstp.reshape(-1, CH)], axis=1)

    h0 = _tc_linear(x, W_pre, b_pre)
    p1, deg, loops = _sc_aggregate(h0, ei3, n_pad, True)
    deg3 = deg.reshape(NC, n_pad, 1)
    loop3 = loops.reshape(NC, n_pad, 1)
    h1 = _tc_mean_linear(p1, h0, deg3, loop3, W1, b1, relu=True,
                         normalize=False)
    (p2,) = _sc_aggregate(h1, ei3, n_pad, False)
    return _tc_mean_linear(p2, h1, deg3, loop3, W2, b2, relu=False,
                           normalize=True)
